# initial kernel scaffold (unmeasured)
import jax
import jax.numpy as jnp
from jax import lax
from jax.experimental import pallas as pl
from jax.experimental.pallas import tpu as pltpu

N_DEV = 4


def kernel(x, w_mat):
    partial = jnp.dot(x, w_mat, preferred_element_type=jnp.float32)
    return _allreduce_quant(partial)


def _allreduce_quant(p):
    m, n = p.shape
    mp = m // N_DEV

    def body(p_ref, out_ref, comm_ref, send_sems, recv_sems, credit_sems):
        my = lax.axis_index("i")
        left = lax.rem(my + (N_DEV - 1), N_DEV)
        right = lax.rem(my + 1, N_DEV)

        def chunk(c):
            return pl.ds(c * mp, mp)

        barrier_sem = pltpu.get_barrier_semaphore()
        for nbr in (left, right):
            pl.semaphore_signal(
                barrier_sem, inc=1,
                device_id=(nbr,), device_id_type=pl.DeviceIdType.MESH,
            )
        pl.semaphore_wait(barrier_sem, 2)

        comm_ref[2] = p_ref[chunk(lax.rem(my + 3, N_DEV)), :]

        for h in range(6):
            dst_slot = h % 2
            src_slot = 2 if h == 0 else (h - 1) % 2
            if h >= 2:
                pl.semaphore_wait(credit_sems.at[dst_slot], 1)
            rdma = pltpu.make_async_remote_copy(
                src_ref=comm_ref.at[src_slot],
                dst_ref=comm_ref.at[dst_slot],
                send_sem=send_sems.at[dst_slot],
                recv_sem=recv_sems.at[dst_slot],
                device_id=(right,),
                device_id_type=pl.DeviceIdType.MESH,
            )
            rdma.start()
            rdma.wait()
            if h >= 1:
                pl.semaphore_signal(
                    credit_sems.at[src_slot], inc=1,
                    device_id=(left,), device_id_type=pl.DeviceIdType.MESH,
                )
            if h < 3:
                c = lax.rem(my + (N_DEV - 2 - h) + N_DEV, N_DEV)
                comm_ref[dst_slot] = (
                    comm_ref[dst_slot] + p_ref[chunk(c), :]
                )
                if h == 2:
                    out_ref[chunk(my), :] = comm_ref[dst_slot]
            else:
                g = h - 3
                c = lax.rem(my + (N_DEV - 1 - g) + N_DEV, N_DEV)
                out_ref[chunk(c), :] = comm_ref[dst_slot]

        y = out_ref[...]
        amax = jnp.max(jnp.abs(y))
        scale = amax / 448.0
        q = (y * (1.0 / scale)).astype(jnp.float8_e4m3fn)
        out_ref[...] = q.astype(jnp.float32) * scale

    return pl.pallas_call(
        body,
        out_shape=jax.ShapeDtypeStruct((m, n), jnp.float32),
        in_specs=[pl.BlockSpec(memory_space=pltpu.VMEM)],
        out_specs=pl.BlockSpec(memory_space=pltpu.VMEM),
        scratch_shapes=[
            pltpu.VMEM((3, mp, n), jnp.float32),
            pltpu.SemaphoreType.DMA((2,)),
            pltpu.SemaphoreType.DMA((2,)),
            pltpu.SemaphoreType.REGULAR((2,)),
        ],
        compiler_params=pltpu.CompilerParams(collective_id=0),
    )(p)


# baseline (device time: 375472 ns/iter reference)
import jax
import jax.numpy as jnp
from jax import lax
from jax.experimental import pallas as pl
from jax.experimental.pallas import tpu as pltpu

N_DEV = 4


def kernel(x, w_mat):
    partial = jnp.dot(x, w_mat, preferred_element_type=jnp.float32)
    return _allreduce_quant(partial)


def _allreduce_quant(p):
    m, n = p.shape
    mp = m // N_DEV
    nh = n // 2

    def body(p_hbm, out_hbm, commA, commB, acc_ref,
             sendA, recvA, sendB, recvB, creditA, creditB, local_sems):
        my = lax.axis_index("i")
        left = lax.rem(my + (N_DEV - 1), N_DEV)
        right = lax.rem(my + 1, N_DEV)

        def rows(c):
            return pl.ds(c * mp, mp)

        A = pl.ds(0, nh)
        B = pl.ds(nh, nh)

        def cmod(k):
            return lax.rem(my + (k + 2 * N_DEV), N_DEV)

        def copy(src, dst, sem_i):
            cp = pltpu.make_async_copy(src, dst, local_sems.at[sem_i])
            cp.start()
            return cp

        barrier_sem = pltpu.get_barrier_semaphore()
        for nbr in (left, right):
            pl.semaphore_signal(
                barrier_sem, inc=1,
                device_id=(nbr,), device_id_type=pl.DeviceIdType.MESH,
            )
        pl.semaphore_wait(barrier_sem, 2)

        st_a = copy(p_hbm.at[rows(cmod(-1)), A], commA.at[2], 0)
        st_b = copy(p_hbm.at[rows(cmod(+1)), B], commB.at[2], 1)
        st_a.wait()
        st_b.wait()

        amax = jnp.float32(0.0)
        pending_stores = []

        for h in range(6):
            ds_ = h % 2
            ss_ = 2 if h == 0 else (h - 1) % 2
            if h >= 2:
                pl.semaphore_wait(creditA.at[ds_], 1)
                pl.semaphore_wait(creditB.at[ds_], 1)
            rdma_a = pltpu.make_async_remote_copy(
                src_ref=commA.at[ss_], dst_ref=commA.at[ds_],
                send_sem=sendA.at[ds_], recv_sem=recvA.at[ds_],
                device_id=(right,), device_id_type=pl.DeviceIdType.MESH,
            )
            rdma_b = pltpu.make_async_remote_copy(
                src_ref=commB.at[ss_], dst_ref=commB.at[ds_],
                send_sem=sendB.at[ds_], recv_sem=recvB.at[ds_],
                device_id=(left,), device_id_type=pl.DeviceIdType.MESH,
            )
            rdma_a.start()
            rdma_b.start()

            loads = []
            if h < 3:
                cA, cB = cmod(-2 - h), cmod(2 + h)
                if h == 1:
                    loads.append(copy(p_hbm.at[rows(cA), A],
                                      acc_ref.at[:, A], 0))
                    loads.append(copy(p_hbm.at[rows(cB), B],
                                      acc_ref.at[:, B], 1))
                else:
                    loads.append(copy(p_hbm.at[rows(cA)], acc_ref, 0))

            rdma_a.wait()
            rdma_b.wait()

            for s in pending_stores:
                s.wait()
            pending_stores = []
            if 1 <= h <= 4:
                pl.semaphore_signal(
                    creditA.at[ss_], inc=1,
                    device_id=(left,), device_id_type=pl.DeviceIdType.MESH,
                )
                pl.semaphore_signal(
                    creditB.at[ss_], inc=1,
                    device_id=(right,), device_id_type=pl.DeviceIdType.MESH,
                )

            if h < 3:
                for ld in loads:
                    ld.wait()
                commA[ds_] = commA[ds_] + acc_ref[:, A]
                commB[ds_] = commB[ds_] + acc_ref[:, B]
                if h == 2:
                    amax = jnp.maximum(amax, jnp.max(jnp.abs(commA[ds_])))
                    amax = jnp.maximum(amax, jnp.max(jnp.abs(commB[ds_])))
                    pending_stores.append(
                        copy(commA.at[ds_], out_hbm.at[rows(my), A], 2))
                    pending_stores.append(
                        copy(commB.at[ds_], out_hbm.at[rows(my), B], 3))
            else:
                g = h - 3
                amax = jnp.maximum(amax, jnp.max(jnp.abs(commA[ds_])))
                amax = jnp.maximum(amax, jnp.max(jnp.abs(commB[ds_])))
                pending_stores.append(
                    copy(commA.at[ds_], out_hbm.at[rows(cmod(-1 - g)), A], 2))
                pending_stores.append(
                    copy(commB.at[ds_], out_hbm.at[rows(cmod(+1 + g)), B], 3))

        for s in pending_stores:
            s.wait()

        scale = amax / 448.0
        inv = 448.0 / amax
        for c in range(N_DEV):
            copy(out_hbm.at[rows(c)], acc_ref, 0).wait()
            q = (acc_ref[...] * inv).astype(jnp.float8_e4m3fn)
            acc_ref[...] = q.astype(jnp.float32) * scale
            copy(acc_ref, out_hbm.at[rows(c)], 0).wait()

    return pl.pallas_call(
        body,
        out_shape=jax.ShapeDtypeStruct((m, n), jnp.float32),
        in_specs=[pl.BlockSpec(memory_space=pltpu.MemorySpace.HBM)],
        out_specs=pl.BlockSpec(memory_space=pltpu.MemorySpace.HBM),
        scratch_shapes=[
            pltpu.VMEM((3, mp, nh), jnp.float32),
            pltpu.VMEM((3, mp, nh), jnp.float32),
            pltpu.VMEM((mp, n), jnp.float32),
            pltpu.SemaphoreType.DMA((2,)),
            pltpu.SemaphoreType.DMA((2,)),
            pltpu.SemaphoreType.DMA((2,)),
            pltpu.SemaphoreType.DMA((2,)),
            pltpu.SemaphoreType.REGULAR((2,)),
            pltpu.SemaphoreType.REGULAR((2,)),
            pltpu.SemaphoreType.DMA((4,)),
        ],
        compiler_params=pltpu.CompilerParams(
            collective_id=0, vmem_limit_bytes=60 * 2**20),
    )(p)


# device time: 346018 ns/iter; 1.0851x vs baseline; 1.0851x over previous
import jax
import jax.numpy as jnp
from jax import lax
from jax.experimental import pallas as pl
from jax.experimental.pallas import tpu as pltpu

N_DEV = 4


def kernel(x, w_mat):
    m, k = x.shape
    _, n = w_mat.shape
    mp = m // N_DEV
    nh = n // 2

    def body(x_hbm, w_ref, out_hbm, xc, commA, commB, acc, keep,
             sendA, recvA, sendB, recvB, creditA, creditB, local_sems):
        my = lax.axis_index("i")
        left = lax.rem(my + (N_DEV - 1), N_DEV)
        right = lax.rem(my + 1, N_DEV)

        def rows(c):
            return pl.ds(c * mp, mp)

        A = pl.ds(0, nh)
        B = pl.ds(nh, nh)

        def cmod(kk):
            return lax.rem(my + (kk + 2 * N_DEV), N_DEV)

        def copy(src, dst, sem_i):
            cp = pltpu.make_async_copy(src, dst, local_sems.at[sem_i])
            cp.start()
            return cp

        def gemm(xc_slot, dst_ref):
            dst_ref[:, A] = jnp.dot(
                xc[xc_slot], w_ref[:, 0:nh],
                preferred_element_type=jnp.float32)
            dst_ref[:, B] = jnp.dot(
                xc[xc_slot], w_ref[:, nh:n],
                preferred_element_type=jnp.float32)

        barrier_sem = pltpu.get_barrier_semaphore()
        for nbr in (left, right):
            pl.semaphore_signal(
                barrier_sem, inc=1,
                device_id=(nbr,), device_id_type=pl.DeviceIdType.MESH,
            )
        pl.semaphore_wait(barrier_sem, 2)

        copy(x_hbm.at[rows(cmod(-1))], xc.at[0], 0).wait()
        commA[2] = jnp.dot(xc[0], w_ref[:, 0:nh],
                           preferred_element_type=jnp.float32)
        keep[:, B] = jnp.dot(xc[0], w_ref[:, nh:n],
                             preferred_element_type=jnp.float32)
        copy(x_hbm.at[rows(cmod(+1))], xc.at[0], 0).wait()
        keep[:, A] = jnp.dot(xc[0], w_ref[:, 0:nh],
                             preferred_element_type=jnp.float32)
        commB[2] = jnp.dot(xc[0], w_ref[:, nh:n],
                           preferred_element_type=jnp.float32)

        amax = jnp.float32(0.0)
        pending = []

        for h in range(6):
            ds_ = h % 2
            ss_ = 2 if h == 0 else (h - 1) % 2
            if h >= 2:
                pl.semaphore_wait(creditA.at[ds_], 1)
                pl.semaphore_wait(creditB.at[ds_], 1)
            rdma_a = pltpu.make_async_remote_copy(
                src_ref=commA.at[ss_], dst_ref=commA.at[ds_],
                send_sem=sendA.at[ds_], recv_sem=recvA.at[ds_],
                device_id=(right,), device_id_type=pl.DeviceIdType.MESH,
            )
            rdma_b = pltpu.make_async_remote_copy(
                src_ref=commB.at[ss_], dst_ref=commB.at[ds_],
                send_sem=sendB.at[ds_], recv_sem=recvB.at[ds_],
                device_id=(left,), device_id_type=pl.DeviceIdType.MESH,
            )
            rdma_a.start()
            rdma_b.start()

            if h == 0:
                ldx = copy(x_hbm.at[rows(cmod(+2))], xc.at[0], 0)
                ldx.wait()
                gemm(0, acc)
            elif h == 1:
                ldx = copy(x_hbm.at[rows(cmod(0))], xc.at[0], 0)
                ldx.wait()
                gemm(0, acc)

            rdma_a.wait()
            rdma_b.wait()

            for cpy in pending:
                cpy.wait()
            pending = []
            if 1 <= h <= 4:
                pl.semaphore_signal(
                    creditA.at[ss_], inc=1,
                    device_id=(left,), device_id_type=pl.DeviceIdType.MESH,
                )
                pl.semaphore_signal(
                    creditB.at[ss_], inc=1,
                    device_id=(right,), device_id_type=pl.DeviceIdType.MESH,
                )

            if h < 3:
                if h == 1:
                    commA[ds_] = commA[ds_] + keep[:, A]
                    commB[ds_] = commB[ds_] + keep[:, B]
                else:
                    commA[ds_] = commA[ds_] + acc[:, A]
                    commB[ds_] = commB[ds_] + acc[:, B]
                if h == 2:
                    amax = jnp.maximum(amax, jnp.max(jnp.abs(commA[ds_])))
                    amax = jnp.maximum(amax, jnp.max(jnp.abs(commB[ds_])))
                    pending.append(copy(commA.at[ds_], acc.at[:, A], 2))
                    pending.append(copy(commB.at[ds_], acc.at[:, B], 3))
            else:
                amax = jnp.maximum(amax, jnp.max(jnp.abs(commA[ds_])))
                amax = jnp.maximum(amax, jnp.max(jnp.abs(commB[ds_])))
                if h == 3:
                    pending.append(copy(commA.at[ds_], keep.at[:, A], 2))
                    pending.append(copy(commB.at[ds_], keep.at[:, B], 3))

        scale = amax / 448.0
        inv = 448.0 / amax

        def qd(v):
            return (v * inv).astype(jnp.float8_e4m3fn).astype(
                jnp.float32) * scale

        acc[...] = qd(acc[...])
        st = [copy(acc, out_hbm.at[rows(my)], 0)]
        commA[0] = qd(commA[0])
        st.append(copy(commA.at[0], out_hbm.at[rows(cmod(+2)), A], 1))
        commB[0] = qd(commB[0])
        st.append(copy(commB.at[0], out_hbm.at[rows(cmod(+2)), B], 2))
        commA[1] = qd(commA[1])
        st.append(copy(commA.at[1], out_hbm.at[rows(cmod(+1)), A], 3))
        commB[1] = qd(commB[1])
        st.append(copy(commB.at[1], out_hbm.at[rows(cmod(-1)), B], 4))
        keep[...] = qd(keep[...])
        st.append(copy(keep.at[:, A], out_hbm.at[rows(cmod(-1)), A], 5))
        st.append(copy(keep.at[:, B], out_hbm.at[rows(cmod(+1)), B], 6))
        for s in st:
            s.wait()

    return pl.pallas_call(
        body,
        out_shape=jax.ShapeDtypeStruct((m, n), jnp.float32),
        in_specs=[
            pl.BlockSpec(memory_space=pltpu.MemorySpace.HBM),
            pl.BlockSpec(memory_space=pltpu.MemorySpace.VMEM),
        ],
        out_specs=pl.BlockSpec(memory_space=pltpu.MemorySpace.HBM),
        scratch_shapes=[
            pltpu.VMEM((1, mp, k), jnp.float32),
            pltpu.VMEM((3, mp, nh), jnp.float32),
            pltpu.VMEM((3, mp, nh), jnp.float32),
            pltpu.VMEM((mp, n), jnp.float32),
            pltpu.VMEM((mp, n), jnp.float32),
            pltpu.SemaphoreType.DMA((2,)),
            pltpu.SemaphoreType.DMA((2,)),
            pltpu.SemaphoreType.DMA((2,)),
            pltpu.SemaphoreType.DMA((2,)),
            pltpu.SemaphoreType.REGULAR((2,)),
            pltpu.SemaphoreType.REGULAR((2,)),
            pltpu.SemaphoreType.DMA((7,)),
        ],
        compiler_params=pltpu.CompilerParams(
            collective_id=0, vmem_limit_bytes=63 * 2**20),
    )(x, w_mat)


# device time: 340391 ns/iter; 1.1031x vs baseline; 1.0165x over previous
import jax
import jax.numpy as jnp
from jax import lax
from jax.experimental import pallas as pl
from jax.experimental.pallas import tpu as pltpu

N_DEV = 4


def kernel(x, w_mat):
    m, k = x.shape
    _, n = w_mat.shape
    mp = m // N_DEV
    nh = n // 2

    def body(x_hbm, w_ref, out_hbm, xc, commA, commB, acc, keep, amax_ring,
             sendA, recvA, sendB, recvB, creditA, creditB,
             tiny_send, tiny_recv, local_sems):
        my = lax.axis_index("i")
        left = lax.rem(my + (N_DEV - 1), N_DEV)
        right = lax.rem(my + 1, N_DEV)

        def rows(c):
            return pl.ds(c * mp, mp)

        A = pl.ds(0, nh)
        B = pl.ds(nh, nh)

        def cmod(kk):
            return lax.rem(my + (kk + 2 * N_DEV), N_DEV)

        def copy(src, dst, sem_i):
            cp = pltpu.make_async_copy(src, dst, local_sems.at[sem_i])
            cp.start()
            return cp

        def load_x(c):
            copy(x_hbm.at[rows(c)], xc.at[0], 0).wait()

        def mk_rdma(comm, ss_, ds_, ssem, rsem, dev):
            return pltpu.make_async_remote_copy(
                src_ref=comm.at[ss_], dst_ref=comm.at[ds_],
                send_sem=ssem.at[ds_], recv_sem=rsem.at[ds_],
                device_id=(dev,), device_id_type=pl.DeviceIdType.MESH,
            )

        def wA():
            return w_ref[:, 0:nh]

        def wB():
            return w_ref[:, nh:n]

        barrier_sem = pltpu.get_barrier_semaphore()
        for nbr in (left, right):
            pl.semaphore_signal(
                barrier_sem, inc=1,
                device_id=(nbr,), device_id_type=pl.DeviceIdType.MESH,
            )
        pl.semaphore_wait(barrier_sem, 2)

        load_x(cmod(-1))
        commA[2] = jnp.dot(xc[0], wA(), preferred_element_type=jnp.float32)
        rdma_a0 = mk_rdma(commA, 2, 0, sendA, recvA, right)
        rdma_a0.start()
        load_x(cmod(+1))
        commB[2] = jnp.dot(xc[0], wB(), preferred_element_type=jnp.float32)
        rdma_b0 = mk_rdma(commB, 2, 0, sendB, recvB, left)
        rdma_b0.start()
        keep[:, A] = jnp.dot(xc[0], wA(), preferred_element_type=jnp.float32)
        load_x(cmod(-1))
        keep[:, B] = jnp.dot(xc[0], wB(), preferred_element_type=jnp.float32)
        load_x(cmod(+2))
        acc[:, A] = jnp.dot(xc[0], wA(), preferred_element_type=jnp.float32)
        acc[:, B] = jnp.dot(xc[0], wB(), preferred_element_type=jnp.float32)

        amax_my = jnp.float32(0.0)
        pending = []
        tiny = None

        for h in range(6):
            ds_ = h % 2
            ss_ = 2 if h == 0 else (h - 1) % 2
            if h == 0:
                rdma_a, rdma_b = rdma_a0, rdma_b0
            else:
                if h >= 2:
                    pl.semaphore_wait(creditA.at[ds_], 1)
                    pl.semaphore_wait(creditB.at[ds_], 1)
                rdma_a = mk_rdma(commA, ss_, ds_, sendA, recvA, right)
                rdma_b = mk_rdma(commB, ss_, ds_, sendB, recvB, left)
                rdma_a.start()
                rdma_b.start()

            if h == 1:
                load_x(cmod(0))
                acc[:, A] = jnp.dot(xc[0], wA(),
                                    preferred_element_type=jnp.float32)
                acc[:, B] = jnp.dot(xc[0], wB(),
                                    preferred_element_type=jnp.float32)
            elif h >= 3:
                j = h - 3
                if j > 0:
                    tiny.wait()
                tiny = pltpu.make_async_remote_copy(
                    src_ref=amax_ring.at[3 if j == 0 else j - 1],
                    dst_ref=amax_ring.at[j],
                    send_sem=tiny_send.at[j], recv_sem=tiny_recv.at[j],
                    device_id=(right,), device_id_type=pl.DeviceIdType.MESH,
                )
                tiny.start()
                if h == 5:
                    tiny.wait()
                    amax = jnp.maximum(amax_my, amax_ring[0, 0, 0])
                    amax = jnp.maximum(amax, amax_ring[1, 0, 0])
                    amax = jnp.maximum(amax, amax_ring[2, 0, 0])
                    scale = amax / 448.0
                    inv = 448.0 / amax

                    def qd(v):
                        return (v * inv).astype(jnp.float8_e4m3fn).astype(
                            jnp.float32) * scale

                    acc[...] = qd(acc[...])
                    st = [copy(acc, out_hbm.at[rows(my)], 0)]
                    keep[...] = qd(keep[...])
                    st.append(
                        copy(keep.at[:, A], out_hbm.at[rows(cmod(-1)), A], 5))
                    st.append(
                        copy(keep.at[:, B], out_hbm.at[rows(cmod(+1)), B], 6))

            rdma_a.wait()
            rdma_b.wait()

            for cpy in pending:
                cpy.wait()
            pending = []
            if 1 <= h <= 4:
                pl.semaphore_signal(
                    creditA.at[ss_], inc=1,
                    device_id=(left,), device_id_type=pl.DeviceIdType.MESH,
                )
                pl.semaphore_signal(
                    creditB.at[ss_], inc=1,
                    device_id=(right,), device_id_type=pl.DeviceIdType.MESH,
                )

            if h < 3:
                if h == 1:
                    commA[ds_] = commA[ds_] + keep[:, A]
                    commB[ds_] = commB[ds_] + keep[:, B]
                else:
                    commA[ds_] = commA[ds_] + acc[:, A]
                    commB[ds_] = commB[ds_] + acc[:, B]
                if h == 2:
                    amax_my = jnp.maximum(
                        jnp.max(jnp.abs(commA[ds_])),
                        jnp.max(jnp.abs(commB[ds_])))
                    amax_ring[3] = jnp.full((8, 128), amax_my,
                                            dtype=jnp.float32)
                    pending.append(copy(commA.at[ds_], acc.at[:, A], 2))
                    pending.append(copy(commB.at[ds_], acc.at[:, B], 3))
            elif h == 3:
                pending.append(copy(commA.at[ds_], keep.at[:, A], 2))
                pending.append(copy(commB.at[ds_], keep.at[:, B], 3))

        commA[0] = qd(commA[0])
        st.append(copy(commA.at[0], out_hbm.at[rows(cmod(+2)), A], 1))
        commB[0] = qd(commB[0])
        st.append(copy(commB.at[0], out_hbm.at[rows(cmod(+2)), B], 2))
        commA[1] = qd(commA[1])
        st.append(copy(commA.at[1], out_hbm.at[rows(cmod(+1)), A], 3))
        commB[1] = qd(commB[1])
        st.append(copy(commB.at[1], out_hbm.at[rows(cmod(-1)), B], 4))
        for s in st:
            s.wait()

    return pl.pallas_call(
        body,
        out_shape=jax.ShapeDtypeStruct((m, n), jnp.float32),
        in_specs=[
            pl.BlockSpec(memory_space=pltpu.MemorySpace.HBM),
            pl.BlockSpec(memory_space=pltpu.MemorySpace.VMEM),
        ],
        out_specs=pl.BlockSpec(memory_space=pltpu.MemorySpace.HBM),
        scratch_shapes=[
            pltpu.VMEM((1, mp, k), jnp.float32),
            pltpu.VMEM((3, mp, nh), jnp.float32),
            pltpu.VMEM((3, mp, nh), jnp.float32),
            pltpu.VMEM((mp, n), jnp.float32),
            pltpu.VMEM((mp, n), jnp.float32),
            pltpu.VMEM((4, 8, 128), jnp.float32),
            pltpu.SemaphoreType.DMA((2,)),
            pltpu.SemaphoreType.DMA((2,)),
            pltpu.SemaphoreType.DMA((2,)),
            pltpu.SemaphoreType.DMA((2,)),
            pltpu.SemaphoreType.REGULAR((2,)),
            pltpu.SemaphoreType.REGULAR((2,)),
            pltpu.SemaphoreType.DMA((3,)),
            pltpu.SemaphoreType.DMA((3,)),
            pltpu.SemaphoreType.DMA((7,)),
        ],
        compiler_params=pltpu.CompilerParams(
            collective_id=0, vmem_limit_bytes=63 * 2**20),
    )(x, w_mat)


# device time: 318709 ns/iter; 1.1781x vs baseline; 1.0680x over previous
import jax
import jax.numpy as jnp
from jax import lax
from jax.experimental import pallas as pl
from jax.experimental.pallas import tpu as pltpu

N_DEV = 4


def kernel(x, w_mat):
    m, k = x.shape
    _, n = w_mat.shape
    mp = m // N_DEV
    hp = mp // 2
    nh = n // 2

    def body(x_hbm, w_ref, out_hbm, xc,
             cAT, cBT, cAU, cBU, acc, keep, amax_ring,
             sAT, rAT, sBT, rBT, sAU, rAU, sBU, rBU,
             crAT, crBT, crAU, crBU,
             tiny_send, tiny_recv, local_sems):
        my = lax.axis_index("i")
        left = lax.rem(my + (N_DEV - 1), N_DEV)
        right = lax.rem(my + 1, N_DEV)

        RT = pl.ds(0, hp)
        RU = pl.ds(hp, hp)
        A = pl.ds(0, nh)
        B = pl.ds(nh, nh)

        def cmod(kk):
            return lax.rem(my + (kk + 2 * N_DEV), N_DEV)

        def rows_s(c, st):
            return pl.ds(c * mp + (0 if st == 0 else hp), hp)

        def copy(src, dst, sem_i):
            cp = pltpu.make_async_copy(src, dst, local_sems.at[sem_i])
            cp.start()
            return cp

        def load_x(c):
            copy(x_hbm.at[pl.ds(c * mp, mp)], xc.at[0], 0).wait()

        def dot(a, b):
            return jnp.dot(a, b, preferred_element_type=jnp.float32)

        T = dict(cA=cAT, cB=cBT, sA=sAT, rA=rAT, sB=sBT, rB=rBT,
                 crA=crAT, crB=crBT, rs=RT, st=0, sh=(1, 2), pend=[])
        U = dict(cA=cAU, cB=cBU, sA=sAU, rA=rAU, sB=sBU, rB=rBU,
                 crA=crAU, crB=crBU, rs=RU, st=1, sh=(3, 4), pend=[])

        def mk_pair(S, ss_, ds_):
            ra = pltpu.make_async_remote_copy(
                src_ref=S["cA"].at[ss_], dst_ref=S["cA"].at[ds_],
                send_sem=S["sA"].at[ds_], recv_sem=S["rA"].at[ds_],
                device_id=(right,), device_id_type=pl.DeviceIdType.MESH,
            )
            rb = pltpu.make_async_remote_copy(
                src_ref=S["cB"].at[ss_], dst_ref=S["cB"].at[ds_],
                send_sem=S["sB"].at[ds_], recv_sem=S["rB"].at[ds_],
                device_id=(left,), device_id_type=pl.DeviceIdType.MESH,
            )
            return ra, rb

        def start_hop(S, h):
            ds_ = h % 2
            ss_ = 2 if h == 0 else (h - 1) % 2
            if h >= 2:
                pl.semaphore_wait(S["crA"].at[ds_], 1)
                pl.semaphore_wait(S["crB"].at[ds_], 1)
            ra, rb = mk_pair(S, ss_, ds_)
            ra.start()
            rb.start()
            S["rdma"] = (ra, rb)

        barrier_sem = pltpu.get_barrier_semaphore()
        for nbr in (left, right):
            pl.semaphore_signal(
                barrier_sem, inc=1,
                device_id=(nbr,), device_id_type=pl.DeviceIdType.MESH,
            )
        pl.semaphore_wait(barrier_sem, 2)

        ta, tb = mk_pair(T, 2, 0)
        ua, ub = mk_pair(U, 2, 0)
        load_x(cmod(-1))
        cAT[2] = dot(xc[0, RT], w_ref[:, A])
        ta.start()
        cAU[2] = dot(xc[0, RU], w_ref[:, A])
        ua.start()
        keep[:, B] = dot(xc[0], w_ref[:, B])
        load_x(cmod(+1))
        cBT[2] = dot(xc[0, RT], w_ref[:, B])
        tb.start()
        cBU[2] = dot(xc[0, RU], w_ref[:, B])
        ub.start()
        keep[:, A] = dot(xc[0], w_ref[:, A])
        T["rdma"] = (ta, tb)
        U["rdma"] = (ua, ub)
        load_x(cmod(+2))
        acc[:, A] = dot(xc[0], w_ref[:, A])
        acc[:, B] = dot(xc[0], w_ref[:, B])

        amax_part = [jnp.float32(0.0)]
        tinys = [None] * 3
        st = []

        def process(S, h):
            ds_ = h % 2
            ss_ = 2 if h == 0 else (h - 1) % 2
            ra, rb = S["rdma"]
            ra.wait()
            rb.wait()
            for cpy in S["pend"]:
                cpy.wait()
            S["pend"] = []
            if 1 <= h <= 4:
                pl.semaphore_signal(
                    S["crA"].at[ss_], inc=1,
                    device_id=(left,), device_id_type=pl.DeviceIdType.MESH,
                )
                pl.semaphore_signal(
                    S["crB"].at[ss_], inc=1,
                    device_id=(right,), device_id_type=pl.DeviceIdType.MESH,
                )
            rs = S["rs"]
            if h < 3:
                src = keep if h == 1 else acc
                S["cA"][ds_] = S["cA"][ds_] + src[rs, A]
                S["cB"][ds_] = S["cB"][ds_] + src[rs, B]
                if h == 2:
                    amax_part[0] = jnp.maximum(
                        amax_part[0],
                        jnp.maximum(jnp.max(jnp.abs(S["cA"][ds_])),
                                    jnp.max(jnp.abs(S["cB"][ds_]))))
                    S["pend"].append(
                        copy(S["cA"].at[ds_], acc.at[rs, A], S["sh"][0]))
                    S["pend"].append(
                        copy(S["cB"].at[ds_], acc.at[rs, B], S["sh"][1]))
            elif h == 3:
                S["pend"].append(
                    copy(S["cA"].at[ds_], keep.at[rs, A], S["sh"][0]))
                S["pend"].append(
                    copy(S["cB"].at[ds_], keep.at[rs, B], S["sh"][1]))
            if h < 5:
                start_hop(S, h + 1)

        for h in range(6):
            if h == 1:
                load_x(cmod(0))
                acc[:, A] = dot(xc[0], w_ref[:, A])
                acc[:, B] = dot(xc[0], w_ref[:, B])
            elif h >= 3:
                j = h - 3
                if j > 0:
                    tinys[j - 1].wait()
                tinys[j] = pltpu.make_async_remote_copy(
                    src_ref=amax_ring.at[3 if j == 0 else j - 1],
                    dst_ref=amax_ring.at[j],
                    send_sem=tiny_send.at[j], recv_sem=tiny_recv.at[j],
                    device_id=(right,), device_id_type=pl.DeviceIdType.MESH,
                )
                tinys[j].start()
                if h == 5:
                    tinys[2].wait()
                    amax = jnp.maximum(amax_part[0], amax_ring[0, 0, 0])
                    amax = jnp.maximum(amax, amax_ring[1, 0, 0])
                    amax = jnp.maximum(amax, amax_ring[2, 0, 0])
                    scale = amax / 448.0
                    inv = 448.0 / amax

                    def qd(v):
                        return (v * inv).astype(jnp.float8_e4m3fn).astype(
                            jnp.float32) * scale

                    acc[...] = qd(acc[...])
                    st.append(copy(acc, out_hbm.at[pl.ds(my * mp, mp)], 5))
                    keep[...] = qd(keep[...])
                    st.append(copy(keep.at[:, A],
                                   out_hbm.at[pl.ds(cmod(-1) * mp, mp), A], 6))
                    st.append(copy(keep.at[:, B],
                                   out_hbm.at[pl.ds(cmod(+1) * mp, mp), B], 7))

            process(T, h)
            if h == 2:
                pass
            process(U, h)
            if h == 2:
                amax_ring[3] = jnp.full((8, 128), amax_part[0],
                                        dtype=jnp.float32)

        for S, sems in ((T, (1, 2, 3, 4)), (U, (8, 9, 10, 11))):
            stt = S["st"]
            S["cA"][0] = qd(S["cA"][0])
            st.append(copy(S["cA"].at[0],
                           out_hbm.at[rows_s(cmod(+2), stt), A], sems[0]))
            S["cB"][0] = qd(S["cB"][0])
            st.append(copy(S["cB"].at[0],
                           out_hbm.at[rows_s(cmod(+2), stt), B], sems[1]))
            S["cA"][1] = qd(S["cA"][1])
            st.append(copy(S["cA"].at[1],
                           out_hbm.at[rows_s(cmod(+1), stt), A], sems[2]))
            S["cB"][1] = qd(S["cB"][1])
            st.append(copy(S["cB"].at[1],
                           out_hbm.at[rows_s(cmod(-1), stt), B], sems[3]))
        for s in st:
            s.wait()

    return pl.pallas_call(
        body,
        out_shape=jax.ShapeDtypeStruct((m, n), jnp.float32),
        in_specs=[
            pl.BlockSpec(memory_space=pltpu.MemorySpace.HBM),
            pl.BlockSpec(memory_space=pltpu.MemorySpace.VMEM),
        ],
        out_specs=pl.BlockSpec(memory_space=pltpu.MemorySpace.HBM),
        scratch_shapes=[
            pltpu.VMEM((1, mp, k), jnp.float32),
            pltpu.VMEM((3, hp, nh), jnp.float32),
            pltpu.VMEM((3, hp, nh), jnp.float32),
            pltpu.VMEM((3, hp, nh), jnp.float32),
            pltpu.VMEM((3, hp, nh), jnp.float32),
            pltpu.VMEM((mp, n), jnp.float32),
            pltpu.VMEM((mp, n), jnp.float32),
            pltpu.VMEM((4, 8, 128), jnp.float32),
            pltpu.SemaphoreType.DMA((2,)),
            pltpu.SemaphoreType.DMA((2,)),
            pltpu.SemaphoreType.DMA((2,)),
            pltpu.SemaphoreType.DMA((2,)),
            pltpu.SemaphoreType.DMA((2,)),
            pltpu.SemaphoreType.DMA((2,)),
            pltpu.SemaphoreType.DMA((2,)),
            pltpu.SemaphoreType.DMA((2,)),
            pltpu.SemaphoreType.REGULAR((2,)),
            pltpu.SemaphoreType.REGULAR((2,)),
            pltpu.SemaphoreType.REGULAR((2,)),
            pltpu.SemaphoreType.REGULAR((2,)),
            pltpu.SemaphoreType.DMA((3,)),
            pltpu.SemaphoreType.DMA((3,)),
            pltpu.SemaphoreType.DMA((12,)),
        ],
        compiler_params=pltpu.CompilerParams(
            collective_id=0, vmem_limit_bytes=63 * 2**20),
    )(x, w_mat)


# device time: 318253 ns/iter; 1.1798x vs baseline; 1.0014x over previous
import jax
import jax.numpy as jnp
from jax import lax
from jax.experimental import pallas as pl
from jax.experimental.pallas import tpu as pltpu

N_DEV = 4


def kernel(x, w_mat):
    m, k = x.shape
    _, n = w_mat.shape
    mp = m // N_DEV
    hp = mp // 2
    nh = n // 2

    def body(x_hbm, w_ref, out_hbm, xc,
             cAT, cBT, cAU, cBU, acc, keep, amax_ring,
             sAT, rAT, sBT, rBT, sAU, rAU, sBU, rBU,
             crAT, crBT, crAU, crBU,
             tiny_send, tiny_recv, local_sems):
        my = lax.axis_index("i")
        left = lax.rem(my + (N_DEV - 1), N_DEV)
        right = lax.rem(my + 1, N_DEV)

        RT = pl.ds(0, hp)
        RU = pl.ds(hp, hp)
        A = pl.ds(0, nh)
        B = pl.ds(nh, nh)

        def cmod(kk):
            return lax.rem(my + (kk + 2 * N_DEV), N_DEV)

        def rows_s(c, st):
            return pl.ds(c * mp + (0 if st == 0 else hp), hp)

        def copy(src, dst, sem_i):
            cp = pltpu.make_async_copy(src, dst, local_sems.at[sem_i])
            cp.start()
            return cp

        def load_x(c):
            copy(x_hbm.at[pl.ds(c * mp, mp)], xc.at[0], 0).wait()

        def dot(a, b):
            return jnp.dot(a, b, preferred_element_type=jnp.float32)

        T = dict(cA=cAT, cB=cBT, sA=sAT, rA=rAT, sB=sBT, rB=rBT,
                 crA=crAT, crB=crBT, rs=RT, st=0, sh=(1, 2), pend=[])
        U = dict(cA=cAU, cB=cBU, sA=sAU, rA=rAU, sB=sBU, rB=rBU,
                 crA=crAU, crB=crBU, rs=RU, st=1, sh=(3, 4), pend=[])

        def mk_pair(S, ss_, ds_):
            ra = pltpu.make_async_remote_copy(
                src_ref=S["cA"].at[ss_], dst_ref=S["cA"].at[ds_],
                send_sem=S["sA"].at[ds_], recv_sem=S["rA"].at[ds_],
                device_id=(right,), device_id_type=pl.DeviceIdType.MESH,
            )
            rb = pltpu.make_async_remote_copy(
                src_ref=S["cB"].at[ss_], dst_ref=S["cB"].at[ds_],
                send_sem=S["sB"].at[ds_], recv_sem=S["rB"].at[ds_],
                device_id=(left,), device_id_type=pl.DeviceIdType.MESH,
            )
            return ra, rb

        def start_hop(S, h):
            ds_ = h % 2
            ss_ = 2 if h == 0 else (h - 1) % 2
            if h >= 2:
                pl.semaphore_wait(S["crA"].at[ds_], 1)
                pl.semaphore_wait(S["crB"].at[ds_], 1)
            ra, rb = mk_pair(S, ss_, ds_)
            ra.start()
            rb.start()
            S["rdma"] = (ra, rb)

        barrier_sem = pltpu.get_barrier_semaphore()
        for nbr in (left, right):
            pl.semaphore_signal(
                barrier_sem, inc=1,
                device_id=(nbr,), device_id_type=pl.DeviceIdType.MESH,
            )
        pl.semaphore_wait(barrier_sem, 2)

        ta, tb = mk_pair(T, 2, 0)
        ua, ub = mk_pair(U, 2, 0)
        load_x(cmod(-1))
        cAT[2] = dot(xc[0, RT], w_ref[:, A])
        ta.start()
        cAU[2] = dot(xc[0, RU], w_ref[:, A])
        ua.start()
        load_x(cmod(+1))
        cBT[2] = dot(xc[0, RT], w_ref[:, B])
        tb.start()
        cBU[2] = dot(xc[0, RU], w_ref[:, B])
        ub.start()
        keep[:, A] = dot(xc[0], w_ref[:, A])
        load_x(cmod(-1))
        keep[:, B] = dot(xc[0], w_ref[:, B])
        T["rdma"] = (ta, tb)
        U["rdma"] = (ua, ub)
        load_x(cmod(+2))
        acc[:, A] = dot(xc[0], w_ref[:, A])
        acc[:, B] = dot(xc[0], w_ref[:, B])

        amax_part = [jnp.float32(0.0)]
        tinys = [None] * 3
        st = []

        def process(S, h):
            ds_ = h % 2
            ss_ = 2 if h == 0 else (h - 1) % 2
            ra, rb = S["rdma"]
            ra.wait()
            rb.wait()
            for cpy in S["pend"]:
                cpy.wait()
            S["pend"] = []
            if 1 <= h <= 4:
                pl.semaphore_signal(
                    S["crA"].at[ss_], inc=1,
                    device_id=(left,), device_id_type=pl.DeviceIdType.MESH,
                )
                pl.semaphore_signal(
                    S["crB"].at[ss_], inc=1,
                    device_id=(right,), device_id_type=pl.DeviceIdType.MESH,
                )
            rs = S["rs"]
            if h < 3:
                src = keep if h == 1 else acc
                S["cA"][ds_] = S["cA"][ds_] + src[rs, A]
                S["cB"][ds_] = S["cB"][ds_] + src[rs, B]
                if h == 2:
                    amax_part[0] = jnp.maximum(
                        amax_part[0],
                        jnp.maximum(jnp.max(jnp.abs(S["cA"][ds_])),
                                    jnp.max(jnp.abs(S["cB"][ds_]))))
                    S["pend"].append(
                        copy(S["cA"].at[ds_], acc.at[rs, A], S["sh"][0]))
                    S["pend"].append(
                        copy(S["cB"].at[ds_], acc.at[rs, B], S["sh"][1]))
            elif h == 3:
                S["pend"].append(
                    copy(S["cA"].at[ds_], keep.at[rs, A], S["sh"][0]))
                S["pend"].append(
                    copy(S["cB"].at[ds_], keep.at[rs, B], S["sh"][1]))
            if h < 5:
                start_hop(S, h + 1)

        for h in range(6):
            if h == 1:
                load_x(cmod(0))
                acc[:, A] = dot(xc[0], w_ref[:, A])
                acc[:, B] = dot(xc[0], w_ref[:, B])
            elif h >= 3:
                j = h - 3
                if j > 0:
                    tinys[j - 1].wait()
                tinys[j] = pltpu.make_async_remote_copy(
                    src_ref=amax_ring.at[3 if j == 0 else j - 1],
                    dst_ref=amax_ring.at[j],
                    send_sem=tiny_send.at[j], recv_sem=tiny_recv.at[j],
                    device_id=(right,), device_id_type=pl.DeviceIdType.MESH,
                )
                tinys[j].start()
                if h == 5:
                    tinys[2].wait()
                    amax = jnp.maximum(amax_part[0], amax_ring[0, 0, 0])
                    amax = jnp.maximum(amax, amax_ring[1, 0, 0])
                    amax = jnp.maximum(amax, amax_ring[2, 0, 0])
                    scale = amax / 448.0
                    inv = 448.0 / amax

                    def qd(v):
                        return (v * inv).astype(jnp.float8_e4m3fn).astype(
                            jnp.float32) * scale

                    acc[...] = qd(acc[...])
                    st.append(copy(acc, out_hbm.at[pl.ds(my * mp, mp)], 5))
                    keep[...] = qd(keep[...])
                    st.append(copy(keep.at[:, A],
                                   out_hbm.at[pl.ds(cmod(-1) * mp, mp), A], 6))
                    st.append(copy(keep.at[:, B],
                                   out_hbm.at[pl.ds(cmod(+1) * mp, mp), B], 7))

            def drain(S, sems):
                stt = S["st"]
                S["cA"][0] = qd(S["cA"][0])
                st.append(copy(S["cA"].at[0],
                               out_hbm.at[rows_s(cmod(+2), stt), A], sems[0]))
                S["cB"][0] = qd(S["cB"][0])
                st.append(copy(S["cB"].at[0],
                               out_hbm.at[rows_s(cmod(+2), stt), B], sems[1]))
                S["cA"][1] = qd(S["cA"][1])
                st.append(copy(S["cA"].at[1],
                               out_hbm.at[rows_s(cmod(+1), stt), A], sems[2]))
                S["cB"][1] = qd(S["cB"][1])
                st.append(copy(S["cB"].at[1],
                               out_hbm.at[rows_s(cmod(-1), stt), B], sems[3]))

            process(T, h)
            if h == 5:
                drain(T, (1, 2, 3, 4))
            process(U, h)
            if h == 2:
                amax_ring[3] = jnp.full((8, 128), amax_part[0],
                                        dtype=jnp.float32)
            elif h == 5:
                drain(U, (8, 9, 10, 11))

        for s in st:
            s.wait()

    return pl.pallas_call(
        body,
        out_shape=jax.ShapeDtypeStruct((m, n), jnp.float32),
        in_specs=[
            pl.BlockSpec(memory_space=pltpu.MemorySpace.HBM),
            pl.BlockSpec(memory_space=pltpu.MemorySpace.VMEM),
        ],
        out_specs=pl.BlockSpec(memory_space=pltpu.MemorySpace.HBM),
        scratch_shapes=[
            pltpu.VMEM((1, mp, k), jnp.float32),
            pltpu.VMEM((3, hp, nh), jnp.float32),
            pltpu.VMEM((3, hp, nh), jnp.float32),
            pltpu.VMEM((3, hp, nh), jnp.float32),
            pltpu.VMEM((3, hp, nh), jnp.float32),
            pltpu.VMEM((mp, n), jnp.float32),
            pltpu.VMEM((mp, n), jnp.float32),
            pltpu.VMEM((4, 8, 128), jnp.float32),
            pltpu.SemaphoreType.DMA((2,)),
            pltpu.SemaphoreType.DMA((2,)),
            pltpu.SemaphoreType.DMA((2,)),
            pltpu.SemaphoreType.DMA((2,)),
            pltpu.SemaphoreType.DMA((2,)),
            pltpu.SemaphoreType.DMA((2,)),
            pltpu.SemaphoreType.DMA((2,)),
            pltpu.SemaphoreType.DMA((2,)),
            pltpu.SemaphoreType.REGULAR((2,)),
            pltpu.SemaphoreType.REGULAR((2,)),
            pltpu.SemaphoreType.REGULAR((2,)),
            pltpu.SemaphoreType.REGULAR((2,)),
            pltpu.SemaphoreType.DMA((3,)),
            pltpu.SemaphoreType.DMA((3,)),
            pltpu.SemaphoreType.DMA((12,)),
        ],
        compiler_params=pltpu.CompilerParams(
            collective_id=0, vmem_limit_bytes=63 * 2**20),
    )(x, w_mat)


# device time: 318156 ns/iter; 1.1802x vs baseline; 1.0003x over previous
import jax
import jax.numpy as jnp
from jax import lax
from jax.experimental import pallas as pl
from jax.experimental.pallas import tpu as pltpu

N_DEV = 4


def kernel(x, w_mat):
    m, k = x.shape
    _, n = w_mat.shape
    mp = m // N_DEV
    hp = mp // 2
    nh = n // 2

    def body(x_hbm, w_ref, out_hbm, xc,
             cAT, cBT, cAU, cBU, acc, keep, amax_ring,
             sAT, rAT, sBT, rBT, sAU, rAU, sBU, rBU,
             crAT, crBT, crAU, crBU,
             tiny_send, tiny_recv, local_sems):
        my = lax.axis_index("i")
        left = lax.rem(my + (N_DEV - 1), N_DEV)
        right = lax.rem(my + 1, N_DEV)

        RT = pl.ds(0, hp)
        RU = pl.ds(hp, hp)
        A = pl.ds(0, nh)
        B = pl.ds(nh, nh)

        def cmod(kk):
            return lax.rem(my + (kk + 2 * N_DEV), N_DEV)

        def rows_s(c, st):
            return pl.ds(c * mp + (0 if st == 0 else hp), hp)

        def copy(src, dst, sem_i):
            cp = pltpu.make_async_copy(src, dst, local_sems.at[sem_i])
            cp.start()
            return cp

        def load_x(c):
            copy(x_hbm.at[pl.ds(c * mp, mp)], xc.at[0], 0).wait()

        def dot(a, b):
            return jnp.dot(a, b, preferred_element_type=jnp.float32)

        T = dict(cA=cAT, cB=cBT, sA=sAT, rA=rAT, sB=sBT, rB=rBT,
                 crA=crAT, crB=crBT, rs=RT, st=0, sh=(1, 2), pend=[])
        U = dict(cA=cAU, cB=cBU, sA=sAU, rA=rAU, sB=sBU, rB=rBU,
                 crA=crAU, crB=crBU, rs=RU, st=1, sh=(3, 4), pend=[])

        def mk_pair(S, ss_, ds_):
            ra = pltpu.make_async_remote_copy(
                src_ref=S["cA"].at[ss_], dst_ref=S["cA"].at[ds_],
                send_sem=S["sA"].at[ds_], recv_sem=S["rA"].at[ds_],
                device_id=(right,), device_id_type=pl.DeviceIdType.MESH,
            )
            rb = pltpu.make_async_remote_copy(
                src_ref=S["cB"].at[ss_], dst_ref=S["cB"].at[ds_],
                send_sem=S["sB"].at[ds_], recv_sem=S["rB"].at[ds_],
                device_id=(left,), device_id_type=pl.DeviceIdType.MESH,
            )
            return ra, rb

        def start_hop(S, h):
            ds_ = h % 2
            ss_ = 2 if h == 0 else (h - 1) % 2
            if h >= 2:
                pl.semaphore_wait(S["crA"].at[ds_], 1)
                pl.semaphore_wait(S["crB"].at[ds_], 1)
            ra, rb = mk_pair(S, ss_, ds_)
            ra.start()
            rb.start()
            S["rdma"] = (ra, rb)

        barrier_sem = pltpu.get_barrier_semaphore()
        for nbr in (left, right):
            pl.semaphore_signal(
                barrier_sem, inc=1,
                device_id=(nbr,), device_id_type=pl.DeviceIdType.MESH,
            )
        pl.semaphore_wait(barrier_sem, 2)

        ta, tb = mk_pair(T, 2, 0)
        ua, ub = mk_pair(U, 2, 0)
        load_x(cmod(-1))
        cAT[2] = dot(xc[0, RT], w_ref[:, A])
        ta.start()
        cAU[2] = dot(xc[0, RU], w_ref[:, A])
        ua.start()
        load_x(cmod(+1))
        cBT[2] = dot(xc[0, RT], w_ref[:, B])
        tb.start()
        cBU[2] = dot(xc[0, RU], w_ref[:, B])
        ub.start()
        keep[:, A] = dot(xc[0], w_ref[:, A])
        load_x(cmod(-1))
        keep[:, B] = dot(xc[0], w_ref[:, B])
        T["rdma"] = (ta, tb)
        U["rdma"] = (ua, ub)
        load_x(cmod(+2))
        acc[:, A] = dot(xc[0], w_ref[:, A])
        acc[:, B] = dot(xc[0], w_ref[:, B])

        amax_part = [jnp.float32(0.0)]
        tinys = [None] * 3
        st = []

        def process(S, h):
            ds_ = h % 2
            ss_ = 2 if h == 0 else (h - 1) % 2
            ra, rb = S["rdma"]
            ra.wait()
            rb.wait()
            for cpy in S["pend"]:
                cpy.wait()
            S["pend"] = []
            if 1 <= h <= 4:
                pl.semaphore_signal(
                    S["crA"].at[ss_], inc=1,
                    device_id=(left,), device_id_type=pl.DeviceIdType.MESH,
                )
                pl.semaphore_signal(
                    S["crB"].at[ss_], inc=1,
                    device_id=(right,), device_id_type=pl.DeviceIdType.MESH,
                )
            rs = S["rs"]
            if h < 3:
                src = keep if h == 1 else acc
                S["cA"][ds_] = S["cA"][ds_] + src[rs, A]
                S["cB"][ds_] = S["cB"][ds_] + src[rs, B]
                if h == 2:
                    amax_part[0] = jnp.maximum(
                        amax_part[0],
                        jnp.maximum(jnp.max(jnp.abs(S["cA"][ds_])),
                                    jnp.max(jnp.abs(S["cB"][ds_]))))
                    S["pend"].append(
                        copy(S["cA"].at[ds_], acc.at[rs, A], S["sh"][0]))
                    S["pend"].append(
                        copy(S["cB"].at[ds_], acc.at[rs, B], S["sh"][1]))
            elif h == 3:
                S["pend"].append(
                    copy(S["cA"].at[ds_], keep.at[rs, A], S["sh"][0]))
                S["pend"].append(
                    copy(S["cB"].at[ds_], keep.at[rs, B], S["sh"][1]))
            if h < 5:
                start_hop(S, h + 1)

        for h in range(6):
            if h == 1:
                load_x(cmod(0))
                acc[:, A] = dot(xc[0], w_ref[:, A])
                acc[:, B] = dot(xc[0], w_ref[:, B])
            elif h >= 3:
                j = h - 3
                if j > 0:
                    tinys[j - 1].wait()
                tinys[j] = pltpu.make_async_remote_copy(
                    src_ref=amax_ring.at[3 if j == 0 else j - 1],
                    dst_ref=amax_ring.at[j],
                    send_sem=tiny_send.at[j], recv_sem=tiny_recv.at[j],
                    device_id=(right,), device_id_type=pl.DeviceIdType.MESH,
                )
                tinys[j].start()
                if h == 5:
                    tinys[2].wait()
                    amax = jnp.maximum(amax_part[0], amax_ring[0, 0, 0])
                    amax = jnp.maximum(amax, amax_ring[1, 0, 0])
                    amax = jnp.maximum(amax, amax_ring[2, 0, 0])
                    scale = amax / 448.0
                    inv = 448.0 / amax

                    def qd(v):
                        return (v * inv).astype(jnp.float8_e4m3fn).astype(
                            jnp.float32) * scale

                    acc[...] = qd(acc[...])
                    st.append(copy(acc, out_hbm.at[pl.ds(my * mp, mp)], 5))
                    keep[...] = qd(keep[...])
                    st.append(copy(keep.at[:, A],
                                   out_hbm.at[pl.ds(cmod(-1) * mp, mp), A], 6))
                    st.append(copy(keep.at[:, B],
                                   out_hbm.at[pl.ds(cmod(+1) * mp, mp), B], 7))

            def drain(S, sems):
                stt = S["st"]
                S["cA"][0] = qd(S["cA"][0])
                st.append(copy(S["cA"].at[0],
                               out_hbm.at[rows_s(cmod(+2), stt), A], sems[0]))
                S["cB"][0] = qd(S["cB"][0])
                st.append(copy(S["cB"].at[0],
                               out_hbm.at[rows_s(cmod(+2), stt), B], sems[1]))
                S["cA"][1] = qd(S["cA"][1])
                st.append(copy(S["cA"].at[1],
                               out_hbm.at[rows_s(cmod(+1), stt), A], sems[2]))
                S["cB"][1] = qd(S["cB"][1])
                st.append(copy(S["cB"].at[1],
                               out_hbm.at[rows_s(cmod(-1), stt), B], sems[3]))

            process(T, h)
            if h == 5:
                drain(T, (1, 2, 3, 4))
            process(U, h)
            if h == 2:
                amax_ring[3] = jnp.full((8, 128), amax_part[0],
                                        dtype=jnp.float32)
            elif h == 5:
                drain(U, (8, 9, 10, 11))

        for s in st:
            s.wait()

    return pl.pallas_call(
        body,
        out_shape=jax.ShapeDtypeStruct((m, n), jnp.float32),
        in_specs=[
            pl.BlockSpec(memory_space=pl.ANY),
            pl.BlockSpec(memory_space=pltpu.MemorySpace.VMEM),
        ],
        out_specs=pl.BlockSpec(memory_space=pl.ANY),
        scratch_shapes=[
            pltpu.VMEM((1, mp, k), jnp.float32),
            pltpu.VMEM((3, hp, nh), jnp.float32),
            pltpu.VMEM((3, hp, nh), jnp.float32),
            pltpu.VMEM((3, hp, nh), jnp.float32),
            pltpu.VMEM((3, hp, nh), jnp.float32),
            pltpu.VMEM((mp, n), jnp.float32),
            pltpu.VMEM((mp, n), jnp.float32),
            pltpu.VMEM((4, 8, 128), jnp.float32),
            pltpu.SemaphoreType.DMA((2,)),
            pltpu.SemaphoreType.DMA((2,)),
            pltpu.SemaphoreType.DMA((2,)),
            pltpu.SemaphoreType.DMA((2,)),
            pltpu.SemaphoreType.DMA((2,)),
            pltpu.SemaphoreType.DMA((2,)),
            pltpu.SemaphoreType.DMA((2,)),
            pltpu.SemaphoreType.DMA((2,)),
            pltpu.SemaphoreType.REGULAR((2,)),
            pltpu.SemaphoreType.REGULAR((2,)),
            pltpu.SemaphoreType.REGULAR((2,)),
            pltpu.SemaphoreType.REGULAR((2,)),
            pltpu.SemaphoreType.DMA((3,)),
            pltpu.SemaphoreType.DMA((3,)),
            pltpu.SemaphoreType.DMA((12,)),
        ],
        compiler_params=pltpu.CompilerParams(
            collective_id=0, vmem_limit_bytes=63 * 2**20),
    )(x, w_mat)
